# Initial kernel scaffold; baseline (speedup 1.0000x reference)
import jax
import jax.numpy as jnp
from jax.experimental import pallas as pl


def kernel(x, table, W, b):
    raise NotImplementedError("write your pallas kernel here")


# speedup vs baseline: 2.4392x; 2.4392x over previous
import functools

import jax
import jax.numpy as jnp
from jax import lax
from jax.experimental import pallas as pl
from jax.experimental.pallas import tpu as pltpu
from jax.experimental.pallas import tpu_sc as plsc

B = 16384
S = 200
D = 64
N_CORES = 2
N_SUBCORES = 16
NW = N_CORES * N_SUBCORES
RPW = B // NW
C = 4
STEPS = RPW // C
HALF = S // 2
LANES = 16
DV = D // LANES


def _sc_pool(x2, table):
  mesh = plsc.VectorSubcoreMesh(core_axis_name="c", subcore_axis_name="s")

  @functools.partial(
      pl.kernel,
      mesh=mesh,
      out_type=jax.ShapeDtypeStruct((B, D), jnp.float32),
      compiler_params=pltpu.CompilerParams(use_tc_tiling_on_sc=False),
      scratch_types=[
          pltpu.VMEM((2 * C, HALF), jnp.int32),
          pltpu.VMEM((C * S, D), jnp.float32),
          pltpu.VMEM((C, D), jnp.float32),
          pltpu.SemaphoreType.DMA,
      ],
  )
  def k(x2_hbm, table_hbm, out_hbm, idx_v, rows_v, pool_v, sem):
    cid = lax.axis_index("c")
    sid = lax.axis_index("s")
    wid = sid * N_CORES + cid
    base = wid * RPW

    def step(t, carry):
      row0 = base + t * C
      pltpu.sync_copy(x2_hbm.at[pl.ds(2 * row0, 2 * C)], idx_v)
      copies = [
          pltpu.async_copy(
              table_hbm.at[idx_v.at[j]],
              rows_v.at[pl.ds(j * HALF, HALF)],
              sem,
          )
          for j in range(2 * C)
      ]
      for cp in copies:
        cp.wait()
      for r in range(C):
        def body(i, accs):
          out = []
          for d in range(DV):
            a = accs[d]
            for u in range(4):
              a = jnp.maximum(
                  a, rows_v[r * S + i * 4 + u, pl.ds(d * LANES, LANES)])
            out.append(a)
          return tuple(out)
        neg = jnp.full((LANES,), -jnp.inf, jnp.float32)
        accs = lax.fori_loop(0, S // 4, body, (neg,) * DV)
        for d in range(DV):
          pool_v[r, pl.ds(d * LANES, LANES)] = accs[d]
      pltpu.sync_copy(pool_v, out_hbm.at[pl.ds(row0, C)])
      return carry

    lax.fori_loop(0, STEPS, step, 0)

  return k(x2, table)


def _matmul(pooled, W, b2):
  N = W.shape[0]
  BM = 1024

  def mm(p_ref, w_ref, b_ref, o_ref):
    o_ref[...] = lax.dot_general(
        p_ref[...], w_ref[...], (((1,), (1,)), ((), ())),
        preferred_element_type=jnp.float32) + b_ref[...]

  return pl.pallas_call(
      mm,
      grid=(B // BM,),
      in_specs=[
          pl.BlockSpec((BM, D), lambda i: (i, 0)),
          pl.BlockSpec((N, D), lambda i: (0, 0)),
          pl.BlockSpec((1, N), lambda i: (0, 0)),
      ],
      out_specs=pl.BlockSpec((BM, N), lambda i: (i, 0)),
      out_shape=jax.ShapeDtypeStruct((B, N), jnp.float32),
  )(pooled, W, b2)


def kernel(x, table, W, b):
  x2 = x.astype(jnp.int32).reshape(2 * B, HALF)
  pooled = _sc_pool(x2, table)
  return _matmul(pooled, W, b.reshape(1, -1))


# speedup vs baseline: 3.2452x; 1.3304x over previous
import functools

import jax
import jax.numpy as jnp
from jax import lax
from jax.experimental import pallas as pl
from jax.experimental.pallas import tpu as pltpu
from jax.experimental.pallas import tpu_sc as plsc

B = 16384
S = 200
D = 64
N_CORES = 2
N_SUBCORES = 16
NW = N_CORES * N_SUBCORES
RPW = B // NW
C = 4
STEPS = RPW // C
HALF = S // 2
NG = 2 * C
LANES = 16
DV = D // LANES


def _sc_pool(x2, table):
  mesh = plsc.VectorSubcoreMesh(core_axis_name="c", subcore_axis_name="s")

  @functools.partial(
      pl.kernel,
      mesh=mesh,
      out_type=jax.ShapeDtypeStruct((B, D), jnp.float32),
      compiler_params=pltpu.CompilerParams(use_tc_tiling_on_sc=False),
      scratch_types=[
          pltpu.VMEM((NG, HALF), jnp.int32),
          pltpu.VMEM((NG, HALF), jnp.int32),
          pltpu.VMEM((C * S, D), jnp.float32),
          pltpu.VMEM((C * S, D), jnp.float32),
          pltpu.VMEM((C, D), jnp.float32),
          pltpu.SemaphoreType.DMA,
          pltpu.SemaphoreType.DMA,
          pltpu.SemaphoreType.DMA,
          pltpu.SemaphoreType.DMA,
      ],
  )
  def k(x2_hbm, table_hbm, out_hbm,
        idx0, idx1, rows0, rows1, pool_v,
        isem0, isem1, rsem0, rsem1):
    cid = lax.axis_index("c")
    sid = lax.axis_index("s")
    wid = sid * N_CORES + cid
    base = wid * RPW
    idx = (idx0, idx1)
    rows = (rows0, rows1)
    isem = (isem0, isem1)
    rsem = (rsem0, rsem1)

    def fire_gathers(nb):
      for j in range(NG):
        pltpu.async_copy(
            table_hbm.at[idx[nb].at[j]],
            rows[nb].at[pl.ds(j * HALF, HALF)],
            rsem[nb],
        )

    def drain_gathers(b):
      for j in range(NG):
        pltpu.make_async_copy(
            table_hbm.at[idx[b].at[j]],
            rows[b].at[pl.ds(j * HALF, HALF)],
            rsem[b],
        ).wait()

    def fire_idx(u, b):
      pltpu.async_copy(
          x2_hbm.at[pl.ds(2 * (base + u * C), NG)], idx[b], isem[b])

    def compute(t, b):
      row0 = base + t * C
      for r in range(C):
        def body(i, accs, r=r, b=b):
          out = []
          for d in range(DV):
            a = accs[d]
            for u in range(4):
              a = jnp.maximum(
                  a, rows[b][r * S + i * 4 + u, pl.ds(d * LANES, LANES)])
            out.append(a)
          return tuple(out)
        neg = jnp.full((LANES,), -jnp.inf, jnp.float32)
        accs = lax.fori_loop(0, S // 4, body, (neg,) * DV)
        for d in range(DV):
          pool_v[r, pl.ds(d * LANES, LANES)] = accs[d]
      pltpu.sync_copy(pool_v, out_hbm.at[pl.ds(row0, C)])

    def phase(t, b):
      nb = 1 - b

      @pl.when(t + 1 < STEPS)
      def _():
        pltpu.make_async_copy(
            x2_hbm.at[pl.ds(0, NG)], idx[nb], isem[nb]).wait()
        fire_gathers(nb)

      drain_gathers(b)

      @pl.when(t + 2 < STEPS)
      def _():
        fire_idx(t + 2, b)

      compute(t, b)

    pltpu.sync_copy(x2_hbm.at[pl.ds(2 * base, NG)], idx[0])
    fire_gathers(0)
    fire_idx(1, 1)

    def outer(i, carry):
      phase(2 * i, 0)
      phase(2 * i + 1, 1)
      return carry

    lax.fori_loop(0, STEPS // 2, outer, 0)

  return k(x2, table)


def _matmul(pooled, W, b2):
  N = W.shape[0]
  BM = 1024

  def mm(p_ref, w_ref, b_ref, o_ref):
    o_ref[...] = lax.dot_general(
        p_ref[...], w_ref[...], (((1,), (1,)), ((), ())),
        preferred_element_type=jnp.float32) + b_ref[...]

  return pl.pallas_call(
      mm,
      grid=(B // BM,),
      in_specs=[
          pl.BlockSpec((BM, D), lambda i: (i, 0)),
          pl.BlockSpec((N, D), lambda i: (0, 0)),
          pl.BlockSpec((1, N), lambda i: (0, 0)),
      ],
      out_specs=pl.BlockSpec((BM, N), lambda i: (i, 0)),
      out_shape=jax.ShapeDtypeStruct((B, N), jnp.float32),
  )(pooled, W, b2)


def kernel(x, table, W, b):
  x2 = x.astype(jnp.int32).reshape(2 * B, HALF)
  pooled = _sc_pool(x2, table)
  return _matmul(pooled, W, b.reshape(1, -1))


# speedup vs baseline: 3.2908x; 1.0141x over previous
import functools

import jax
import jax.numpy as jnp
from jax import lax
from jax.experimental import pallas as pl
from jax.experimental.pallas import tpu as pltpu
from jax.experimental.pallas import tpu_sc as plsc

B = 16384
S = 200
D = 64
N_CORES = 2
N_SUBCORES = 16
NW = N_CORES * N_SUBCORES
RPW = B // NW
C = 4
STEPS = RPW // C
HALF = S // 2
NG = 2 * C
LANES = 16
DV = D // LANES


def _sc_pool(x1, table):
  mesh = plsc.VectorSubcoreMesh(core_axis_name="c", subcore_axis_name="s")

  @functools.partial(
      pl.kernel,
      mesh=mesh,
      out_type=jax.ShapeDtypeStruct((B, D), jnp.float32),
      compiler_params=pltpu.CompilerParams(use_tc_tiling_on_sc=False),
      scratch_types=[
          pltpu.VMEM((NG * HALF,), jnp.int32),
          pltpu.VMEM((NG * HALF,), jnp.int32),
          pltpu.VMEM((C * S, D), jnp.float32),
          pltpu.VMEM((C * S, D), jnp.float32),
          pltpu.VMEM((C, D), jnp.float32),
          pltpu.SemaphoreType.DMA,
          pltpu.SemaphoreType.DMA,
          pltpu.SemaphoreType.DMA,
          pltpu.SemaphoreType.DMA,
      ],
  )
  def k(x1_hbm, table_hbm, out_hbm,
        idx0, idx1, rows0, rows1, pool_v,
        isem0, isem1, rsem0, rsem1):
    cid = lax.axis_index("c")
    sid = lax.axis_index("s")
    wid = sid * N_CORES + cid
    base = wid * RPW
    idx = (idx0, idx1)
    rows = (rows0, rows1)
    isem = (isem0, isem1)
    rsem = (rsem0, rsem1)

    def fire_gathers(nb):
      pltpu.async_copy(
          table_hbm.at[idx[nb]],
          rows[nb],
          rsem[nb],
      )

    def drain_gathers(b):
      pltpu.make_async_copy(
          table_hbm.at[idx[b]],
          rows[b],
          rsem[b],
      ).wait()

    def fire_idx(u, b):
      pltpu.async_copy(
          x1_hbm.at[pl.ds(S * (base + u * C), NG * HALF)], idx[b], isem[b])

    def compute(t, b):
      row0 = base + t * C
      for r in range(C):
        def body(i, accs, r=r, b=b):
          out = []
          for d in range(DV):
            a = accs[d]
            for u in range(4):
              a = jnp.maximum(
                  a, rows[b][r * S + i * 4 + u, pl.ds(d * LANES, LANES)])
            out.append(a)
          return tuple(out)
        neg = jnp.full((LANES,), -jnp.inf, jnp.float32)
        accs = lax.fori_loop(0, S // 4, body, (neg,) * DV)
        for d in range(DV):
          pool_v[r, pl.ds(d * LANES, LANES)] = accs[d]
      pltpu.sync_copy(pool_v, out_hbm.at[pl.ds(row0, C)])

    def phase(t, b):
      nb = 1 - b

      @pl.when(t + 1 < STEPS)
      def _():
        pltpu.make_async_copy(
            x1_hbm.at[pl.ds(0, NG * HALF)], idx[nb], isem[nb]).wait()
        fire_gathers(nb)

      drain_gathers(b)

      @pl.when(t + 2 < STEPS)
      def _():
        fire_idx(t + 2, b)

      compute(t, b)

    pltpu.sync_copy(x1_hbm.at[pl.ds(S * base, NG * HALF)], idx[0])
    fire_gathers(0)
    fire_idx(1, 1)

    def outer(i, carry):
      phase(2 * i, 0)
      phase(2 * i + 1, 1)
      return carry

    lax.fori_loop(0, STEPS // 2, outer, 0)

  return k(x1, table)


def _matmul(pooled, W, b2):
  N = W.shape[0]
  BM = 1024

  def mm(p_ref, w_ref, b_ref, o_ref):
    o_ref[...] = lax.dot_general(
        p_ref[...], w_ref[...], (((1,), (1,)), ((), ())),
        preferred_element_type=jnp.float32) + b_ref[...]

  return pl.pallas_call(
      mm,
      grid=(B // BM,),
      in_specs=[
          pl.BlockSpec((BM, D), lambda i: (i, 0)),
          pl.BlockSpec((N, D), lambda i: (0, 0)),
          pl.BlockSpec((1, N), lambda i: (0, 0)),
      ],
      out_specs=pl.BlockSpec((BM, N), lambda i: (i, 0)),
      out_shape=jax.ShapeDtypeStruct((B, N), jnp.float32),
  )(pooled, W, b2)


def kernel(x, table, W, b):
  x1 = x.astype(jnp.int32).reshape(B * S)
  pooled = _sc_pool(x1, table)
  return _matmul(pooled, W, b.reshape(1, -1))
